# baseline (speedup 1.0000x reference)
import functools

import jax
import jax.numpy as jnp
from jax import lax
from jax.experimental import pallas as pl
from jax.experimental.pallas import tpu as pltpu
from jax.experimental.pallas import tpu_sc as plsc

N = 10000
E = 320000
DH = 128
B = 64
C = 10

NC = 2
NS = 16
NW = NC * NS
K = 128
CHA = 48
CHB = 112
CH_MAX = max(CHA, CHB)
TOTCH = NS * (CHA + CHB)
TOTCH_PAD = TOTCH + CH_MAX
DEG_CH = TOTCH // NW
E_PAD = TOTCH * K
NPAD = 10240
RPT = NPAD // NS

_mesh = plsc.VectorSubcoreMesh(core_axis_name="c", subcore_axis_name="s")
_sc_params = pltpu.CompilerParams(needs_layout_passes=False)


@functools.partial(
    pl.kernel,
    out_type=jax.ShapeDtypeStruct((NC, NPAD), jnp.float32),
    scratch_types=[
        pltpu.VMEM((DEG_CH, K), jnp.int32),
        pltpu.VMEM((NPAD,), jnp.float32),
        pltpu.VMEM((NS, RPT), jnp.float32),
        pltpu.VMEM_SHARED((NS, NPAD), jnp.float32),
    ],
    mesh=_mesh,
    compiler_params=_sc_params,
)
def _deg_kernel(dst_hbm, out_hbm, dst_v, hist_v, red_v, shared):
    c = lax.axis_index("c")
    s = lax.axis_index("s")
    wid = c * NS + s
    pltpu.sync_copy(dst_hbm.at[pl.ds(wid * DEG_CH, DEG_CH)], dst_v)

    def zero_body(i, _):
        hist_v[pl.ds(i * 16, 16)] = jnp.zeros((16,), jnp.float32)
        return 0

    lax.fori_loop(0, NPAD // 16, zero_body, 0)

    ones = jnp.ones((16,), jnp.float32)

    def hist_body(j, _):
        for k in range(K // 16):
            idx = dst_v[j, pl.ds(k * 16, 16)]
            plsc.addupdate_scatter(hist_v, [idx], ones)
        return 0

    lax.fori_loop(0, DEG_CH, hist_body, 0)

    pltpu.sync_copy(hist_v, shared.at[s])
    plsc.subcore_barrier()
    pltpu.sync_copy(shared.at[:, pl.ds(s * RPT, RPT)], red_v)

    def red_body(k, _):
        v = red_v[0, pl.ds(k * 16, 16)]
        for r in range(1, NS):
            v = v + red_v[r, pl.ds(k * 16, 16)]
        hist_v[pl.ds(k * 16, 16)] = v
        return 0

    lax.fori_loop(0, RPT // 16, red_body, 0)
    pltpu.sync_copy(hist_v.at[pl.ds(0, RPT)], out_hbm.at[c, pl.ds(s * RPT, RPT)])


@functools.partial(
    pl.kernel,
    out_type=jax.ShapeDtypeStruct((NC, NPAD, DH), jnp.float32),
    scratch_types=[
        pltpu.VMEM((CH_MAX, K), jnp.int32),
        pltpu.VMEM((2, K), jnp.int32),
        pltpu.VMEM((K, DH), jnp.float32),
        pltpu.VMEM((K, DH), jnp.float32),
        pltpu.VMEM_SHARED((NPAD, DH), jnp.float32),
        pltpu.SemaphoreType.DMA,
        pltpu.SemaphoreType.DMA,
        pltpu.SemaphoreType.DMA,
    ],
    mesh=_mesh,
    compiler_params=_sc_params,
)
def _scatter_kernel(hws_hbm, src_hbm, dst_hbm, out_hbm,
                    src_v, didx, buf0, buf1, acc, sem0, sem1, dsem):
    c = lax.axis_index("c")
    s = lax.axis_index("s")
    chunk0 = jnp.where(c == 0, s * CHA, NS * CHA + s * CHB)
    trips = jnp.where(c == 0, CHA // 2, CHB // 2)
    pltpu.sync_copy(src_hbm.at[pl.ds(chunk0, CH_MAX)], src_v)

    def zero_body(i, _):
        for k in range(DH // 16):
            buf0[i, pl.ds(k * 16, 16)] = jnp.zeros((16,), jnp.float32)
        return 0

    lax.fori_loop(0, K, zero_body, 0)
    base = s * RPT
    for i in range(RPT // K):
        pltpu.sync_copy(buf0, acc.at[pl.ds(base + i * K, K)])
    plsc.subcore_barrier()

    def body(i, _):
        g = 2 * i
        c0 = pltpu.async_copy(hws_hbm.at[src_v.at[g]], buf0, sem0)
        c1 = pltpu.async_copy(hws_hbm.at[src_v.at[g + 1]], buf1, sem1)
        cd = pltpu.async_copy(dst_hbm.at[pl.ds(chunk0 + g, 2)], didx, dsem)
        cd.wait()
        c0.wait()
        pltpu.sync_copy(buf0, acc.at[didx.at[0]], add=True)
        c1.wait()
        pltpu.sync_copy(buf1, acc.at[didx.at[1]], add=True)
        return 0

    lax.fori_loop(0, trips, body, 0)
    plsc.subcore_barrier()
    pltpu.sync_copy(acc.at[pl.ds(base, RPT)], out_hbm.at[c, pl.ds(base, RPT)])


def _tc0_body(x_ref, w_ref, degp_ref, dinv_ref, hws_ref):
    deg = degp_ref[0] + degp_ref[1] + 1.0
    dinv = lax.rsqrt(deg)
    dinv_ref[...] = dinv
    hw = jnp.dot(x_ref[...], w_ref[...], preferred_element_type=jnp.float32)
    hws_ref[...] = hw * dinv[0:N]


_tc0 = pl.pallas_call(
    _tc0_body,
    out_shape=(
        jax.ShapeDtypeStruct((NPAD, 1), jnp.float32),
        jax.ShapeDtypeStruct((N, DH), jnp.float32),
    ),
)


def _tc_mid_body(acc_ref, hws_ref, dinv_ref, b_ref, w_ref, out_ref):
    dinv = dinv_ref[0:N]
    h = dinv * (acc_ref[0, 0:N] + acc_ref[1, 0:N] + hws_ref[...]) + b_ref[...]
    h = jnp.maximum(h, 0.0)
    hw = jnp.dot(h, w_ref[...], preferred_element_type=jnp.float32)
    out_ref[...] = hw * dinv


_tc_mid = pl.pallas_call(
    _tc_mid_body,
    out_shape=jax.ShapeDtypeStruct((N, DH), jnp.float32),
)


def _tc_fin_body(acc_ref, hws_ref, dinv_ref, b_ref, batch_ref,
                 l1w_ref, l1b_ref, l2w_ref, l2b_ref, logp_ref, logits_ref):
    dinv = dinv_ref[0:N]
    h = dinv * (acc_ref[0, 0:N] + acc_ref[1, 0:N] + hws_ref[...]) + b_ref[...]
    h = jnp.maximum(h, 0.0)
    classes = lax.broadcasted_iota(jnp.int32, (B, N), 0)
    oht = (batch_ref[...] == classes).astype(jnp.float32)
    sums = jnp.dot(oht, h, preferred_element_type=jnp.float32)
    cnts = jnp.dot(oht, jnp.ones((N, 1), jnp.float32),
                   preferred_element_type=jnp.float32)
    pooled = sums / jnp.maximum(cnts, 1.0)
    h1 = jnp.dot(pooled, l1w_ref[...], preferred_element_type=jnp.float32)
    h1 = jnp.maximum(h1 + l1b_ref[...], 0.0)
    logits = jnp.dot(h1, l2w_ref[...],
                     preferred_element_type=jnp.float32) + l2b_ref[...]
    m = jnp.max(logits, axis=1, keepdims=True)
    lse = jnp.log(jnp.sum(jnp.exp(logits - m), axis=1, keepdims=True)) + m
    logp_ref[...] = logits - lse
    logits_ref[...] = logits


_tc_fin = pl.pallas_call(
    _tc_fin_body,
    out_shape=(
        jax.ShapeDtypeStruct((B, C), jnp.float32),
        jax.ShapeDtypeStruct((B, C), jnp.float32),
    ),
)


def kernel(x, edge_index, batch, W0, b0, W1, b1, W2, b2,
           lin1_w, lin1_b, lin2_w, lin2_b):
    pad = TOTCH_PAD * K - E
    src3 = jnp.concatenate(
        [edge_index[0], jnp.zeros((pad,), jnp.int32)]).reshape(TOTCH_PAD, K)
    dst3 = jnp.concatenate(
        [edge_index[1], jnp.full((pad,), N, jnp.int32)]).reshape(TOTCH_PAD, K)

    degp = _deg_kernel(dst3).reshape(NC, NPAD, 1)
    dinv, hws = _tc0(x, W0, degp)
    acc = _scatter_kernel(hws, src3, dst3)
    hws = _tc_mid(acc, hws, dinv, b0.reshape(1, DH), W1)
    acc = _scatter_kernel(hws, src3, dst3)
    hws = _tc_mid(acc, hws, dinv, b1.reshape(1, DH), W2)
    acc = _scatter_kernel(hws, src3, dst3)
    logp, logits = _tc_fin(acc, hws, dinv, b2.reshape(1, DH),
                           batch.reshape(1, N).astype(jnp.int32),
                           lin1_w, lin1_b.reshape(1, DH),
                           lin2_w, lin2_b.reshape(1, C))
    return (logp, logits, logits)


# speedup vs baseline: 1.1915x; 1.1915x over previous
import functools

import jax
import jax.numpy as jnp
from jax import lax
from jax.experimental import pallas as pl
from jax.experimental.pallas import tpu as pltpu
from jax.experimental.pallas import tpu_sc as plsc

N = 10000
E = 320000
DH = 128
B = 64
C = 10

NC = 2
NS = 16
NW = NC * NS
K = 128
CHA = 112
CHB = 48
CH_MAX = max(CHA, CHB)
TOTCH = NS * (CHA + CHB)
TOTCH_PAD = TOTCH + CH_MAX
DEG_CH = TOTCH // NW
E_PAD = TOTCH * K
NPAD = 10240
RPT = NPAD // NS

_mesh = plsc.VectorSubcoreMesh(core_axis_name="c", subcore_axis_name="s")
_sc_params = pltpu.CompilerParams(needs_layout_passes=False)


@functools.partial(
    pl.kernel,
    out_type=jax.ShapeDtypeStruct((NC, NPAD), jnp.float32),
    scratch_types=[
        pltpu.VMEM((DEG_CH, K), jnp.int32),
        pltpu.VMEM((NPAD,), jnp.float32),
        pltpu.VMEM((NS, RPT), jnp.float32),
        pltpu.VMEM_SHARED((NS, NPAD), jnp.float32),
    ],
    mesh=_mesh,
    compiler_params=_sc_params,
)
def _deg_kernel(dst_hbm, out_hbm, dst_v, hist_v, red_v, shared):
    c = lax.axis_index("c")
    s = lax.axis_index("s")
    wid = c * NS + s
    pltpu.sync_copy(dst_hbm.at[pl.ds(wid * DEG_CH, DEG_CH)], dst_v)

    def zero_body(i, _):
        hist_v[pl.ds(i * 16, 16)] = jnp.zeros((16,), jnp.float32)
        return 0

    lax.fori_loop(0, NPAD // 16, zero_body, 0)

    ones = jnp.ones((16,), jnp.float32)

    def hist_body(j, _):
        for k in range(K // 16):
            idx = dst_v[j, pl.ds(k * 16, 16)]
            plsc.addupdate_scatter(hist_v, [idx], ones)
        return 0

    lax.fori_loop(0, DEG_CH, hist_body, 0)

    pltpu.sync_copy(hist_v, shared.at[s])
    plsc.subcore_barrier()
    pltpu.sync_copy(shared.at[:, pl.ds(s * RPT, RPT)], red_v)

    def red_body(k, _):
        v = red_v[0, pl.ds(k * 16, 16)]
        for r in range(1, NS):
            v = v + red_v[r, pl.ds(k * 16, 16)]
        hist_v[pl.ds(k * 16, 16)] = v
        return 0

    lax.fori_loop(0, RPT // 16, red_body, 0)
    pltpu.sync_copy(hist_v.at[pl.ds(0, RPT)], out_hbm.at[c, pl.ds(s * RPT, RPT)])


@functools.partial(
    pl.kernel,
    out_type=jax.ShapeDtypeStruct((NC, NPAD, DH), jnp.float32),
    scratch_types=[
        pltpu.VMEM((CH_MAX, K), jnp.int32),
        pltpu.VMEM((2, K), jnp.int32),
        pltpu.VMEM((K, DH), jnp.float32),
        pltpu.VMEM((K, DH), jnp.float32),
        pltpu.VMEM_SHARED((NPAD, DH), jnp.float32),
        pltpu.SemaphoreType.DMA,
        pltpu.SemaphoreType.DMA,
        pltpu.SemaphoreType.DMA,
    ],
    mesh=_mesh,
    compiler_params=_sc_params,
)
def _scatter_kernel(hws_hbm, src_hbm, dst_hbm, out_hbm,
                    src_v, didx, buf0, buf1, acc, sem0, sem1, dsem):
    c = lax.axis_index("c")
    s = lax.axis_index("s")
    chunk0 = jnp.where(c == 0, s * CHA, NS * CHA + s * CHB)
    trips = jnp.where(c == 0, CHA // 2, CHB // 2)
    pltpu.sync_copy(src_hbm.at[pl.ds(chunk0, CH_MAX)], src_v)

    def zero_body(i, _):
        for k in range(DH // 16):
            buf0[i, pl.ds(k * 16, 16)] = jnp.zeros((16,), jnp.float32)
        return 0

    lax.fori_loop(0, K, zero_body, 0)
    base = s * RPT
    for i in range(RPT // K):
        pltpu.sync_copy(buf0, acc.at[pl.ds(base + i * K, K)])
    plsc.subcore_barrier()

    def body(i, _):
        g = 2 * i
        c0 = pltpu.async_copy(hws_hbm.at[src_v.at[g]], buf0, sem0)
        c1 = pltpu.async_copy(hws_hbm.at[src_v.at[g + 1]], buf1, sem1)
        cd = pltpu.async_copy(dst_hbm.at[pl.ds(chunk0 + g, 2)], didx, dsem)
        cd.wait()
        c0.wait()
        pltpu.sync_copy(buf0, acc.at[didx.at[0]], add=True)
        c1.wait()
        pltpu.sync_copy(buf1, acc.at[didx.at[1]], add=True)
        return 0

    lax.fori_loop(0, trips, body, 0)
    plsc.subcore_barrier()
    pltpu.sync_copy(acc.at[pl.ds(base, RPT)], out_hbm.at[c, pl.ds(base, RPT)])


def _tc0_body(x_ref, w_ref, degp_ref, dinv_ref, hws_ref):
    deg = degp_ref[0] + degp_ref[1] + 1.0
    dinv = lax.rsqrt(deg)
    dinv_ref[...] = dinv
    hw = jnp.dot(x_ref[...], w_ref[...], preferred_element_type=jnp.float32)
    hws_ref[...] = hw * dinv[0:N]


_tc0 = pl.pallas_call(
    _tc0_body,
    out_shape=(
        jax.ShapeDtypeStruct((NPAD, 1), jnp.float32),
        jax.ShapeDtypeStruct((N, DH), jnp.float32),
    ),
)


def _tc_mid_body(acc_ref, hws_ref, dinv_ref, b_ref, w_ref, out_ref):
    dinv = dinv_ref[0:N]
    h = dinv * (acc_ref[0, 0:N] + acc_ref[1, 0:N] + hws_ref[...]) + b_ref[...]
    h = jnp.maximum(h, 0.0)
    hw = jnp.dot(h, w_ref[...], preferred_element_type=jnp.float32)
    out_ref[...] = hw * dinv


_tc_mid = pl.pallas_call(
    _tc_mid_body,
    out_shape=jax.ShapeDtypeStruct((N, DH), jnp.float32),
)


def _tc_fin_body(acc_ref, hws_ref, dinv_ref, b_ref, batch_ref,
                 l1w_ref, l1b_ref, l2w_ref, l2b_ref, logp_ref, logits_ref):
    dinv = dinv_ref[0:N]
    h = dinv * (acc_ref[0, 0:N] + acc_ref[1, 0:N] + hws_ref[...]) + b_ref[...]
    h = jnp.maximum(h, 0.0)
    classes = lax.broadcasted_iota(jnp.int32, (B, N), 0)
    oht = (batch_ref[...] == classes).astype(jnp.float32)
    sums = jnp.dot(oht, h, preferred_element_type=jnp.float32)
    cnts = jnp.dot(oht, jnp.ones((N, 1), jnp.float32),
                   preferred_element_type=jnp.float32)
    pooled = sums / jnp.maximum(cnts, 1.0)
    h1 = jnp.dot(pooled, l1w_ref[...], preferred_element_type=jnp.float32)
    h1 = jnp.maximum(h1 + l1b_ref[...], 0.0)
    logits = jnp.dot(h1, l2w_ref[...],
                     preferred_element_type=jnp.float32) + l2b_ref[...]
    m = jnp.max(logits, axis=1, keepdims=True)
    lse = jnp.log(jnp.sum(jnp.exp(logits - m), axis=1, keepdims=True)) + m
    logp_ref[...] = logits - lse
    logits_ref[...] = logits


_tc_fin = pl.pallas_call(
    _tc_fin_body,
    out_shape=(
        jax.ShapeDtypeStruct((B, C), jnp.float32),
        jax.ShapeDtypeStruct((B, C), jnp.float32),
    ),
)


def kernel(x, edge_index, batch, W0, b0, W1, b1, W2, b2,
           lin1_w, lin1_b, lin2_w, lin2_b):
    pad = TOTCH_PAD * K - E
    src3 = jnp.concatenate(
        [edge_index[0], jnp.zeros((pad,), jnp.int32)]).reshape(TOTCH_PAD, K)
    dst3 = jnp.concatenate(
        [edge_index[1], jnp.full((pad,), N, jnp.int32)]).reshape(TOTCH_PAD, K)

    degp = _deg_kernel(dst3).reshape(NC, NPAD, 1)
    dinv, hws = _tc0(x, W0, degp)
    acc = _scatter_kernel(hws, src3, dst3)
    hws = _tc_mid(acc, hws, dinv, b0.reshape(1, DH), W1)
    acc = _scatter_kernel(hws, src3, dst3)
    hws = _tc_mid(acc, hws, dinv, b1.reshape(1, DH), W2)
    acc = _scatter_kernel(hws, src3, dst3)
    logp, logits = _tc_fin(acc, hws, dinv, b2.reshape(1, DH),
                           batch.reshape(1, N).astype(jnp.int32),
                           lin1_w, lin1_b.reshape(1, DH),
                           lin2_w, lin2_b.reshape(1, C))
    return (logp, logits, logits)


# speedup vs baseline: 1.7736x; 1.4886x over previous
import functools

import jax
import jax.numpy as jnp
from jax import lax
from jax.experimental import pallas as pl
from jax.experimental.pallas import tpu as pltpu
from jax.experimental.pallas import tpu_sc as plsc

N = 10000
E = 320000
DH = 128
B = 64
C = 10

NC = 2
NS = 16
NW = NC * NS
K = 128
CHA = 80
CHB = 80
CH_MAX = max(CHA, CHB)
TOTCH = NS * (CHA + CHB)
TOTCH_PAD = TOTCH + CH_MAX
DEG_CH = TOTCH // NW
E_PAD = TOTCH * K
NPAD = 10240
RPT = NPAD // NS

_mesh = plsc.VectorSubcoreMesh(core_axis_name="c", subcore_axis_name="s")
_sc_params = pltpu.CompilerParams(needs_layout_passes=False)


@functools.partial(
    pl.kernel,
    out_type=jax.ShapeDtypeStruct((NC, NPAD), jnp.float32),
    scratch_types=[
        pltpu.VMEM((DEG_CH, K), jnp.int32),
        pltpu.VMEM((NPAD,), jnp.float32),
        pltpu.VMEM((NS, RPT), jnp.float32),
        pltpu.VMEM_SHARED((NS, NPAD), jnp.float32),
    ],
    mesh=_mesh,
    compiler_params=_sc_params,
)
def _deg_kernel(dst_hbm, out_hbm, dst_v, hist_v, red_v, shared):
    c = lax.axis_index("c")
    s = lax.axis_index("s")
    wid = c * NS + s
    pltpu.sync_copy(dst_hbm.at[pl.ds(wid * DEG_CH, DEG_CH)], dst_v)

    def zero_body(i, _):
        hist_v[pl.ds(i * 16, 16)] = jnp.zeros((16,), jnp.float32)
        return 0

    lax.fori_loop(0, NPAD // 16, zero_body, 0)

    ones = jnp.ones((16,), jnp.float32)

    def hist_body(j, _):
        for k in range(K // 16):
            idx = dst_v[j, pl.ds(k * 16, 16)]
            plsc.addupdate_scatter(hist_v, [idx], ones)
        return 0

    lax.fori_loop(0, DEG_CH, hist_body, 0)

    pltpu.sync_copy(hist_v, shared.at[s])
    plsc.subcore_barrier()
    pltpu.sync_copy(shared.at[:, pl.ds(s * RPT, RPT)], red_v)

    def red_body(k, _):
        v = red_v[0, pl.ds(k * 16, 16)]
        for r in range(1, NS):
            v = v + red_v[r, pl.ds(k * 16, 16)]
        hist_v[pl.ds(k * 16, 16)] = v
        return 0

    lax.fori_loop(0, RPT // 16, red_body, 0)
    pltpu.sync_copy(hist_v.at[pl.ds(0, RPT)], out_hbm.at[c, pl.ds(s * RPT, RPT)])


@functools.partial(
    pl.kernel,
    out_type=jax.ShapeDtypeStruct((NC, NPAD, DH), jnp.bfloat16),
    scratch_types=[
        pltpu.VMEM((CH_MAX, K), jnp.int32),
        pltpu.VMEM((2, K), jnp.int32),
        pltpu.VMEM((K, DH), jnp.bfloat16),
        pltpu.VMEM((K, DH), jnp.bfloat16),
        pltpu.VMEM_SHARED((NPAD, DH), jnp.bfloat16),
        pltpu.SemaphoreType.DMA,
        pltpu.SemaphoreType.DMA,
        pltpu.SemaphoreType.DMA,
    ],
    mesh=_mesh,
    compiler_params=pltpu.CompilerParams(
        needs_layout_passes=False, use_tc_tiling_on_sc=False),
)
def _scatter_kernel(hws_hbm, src_hbm, dst_hbm, out_hbm,
                    src_v, didx, buf0, buf1, acc, sem0, sem1, dsem):
    c = lax.axis_index("c")
    s = lax.axis_index("s")
    chunk0 = jnp.where(c == 0, s * CHA, NS * CHA + s * CHB)
    trips = jnp.where(c == 0, CHA // 2, CHB // 2)
    pltpu.sync_copy(src_hbm.at[pl.ds(chunk0, CH_MAX)], src_v)

    def zero_body(i, _):
        for k in range(DH // 32):
            buf0[i, pl.ds(k * 32, 32)] = jnp.zeros((32,), jnp.bfloat16)
        return 0

    lax.fori_loop(0, K, zero_body, 0)
    base = s * RPT
    for i in range(RPT // K):
        pltpu.sync_copy(buf0, acc.at[pl.ds(base + i * K, K)])
    plsc.subcore_barrier()

    def body(i, _):
        g = 2 * i
        c0 = pltpu.async_copy(hws_hbm.at[src_v.at[g]], buf0, sem0)
        c1 = pltpu.async_copy(hws_hbm.at[src_v.at[g + 1]], buf1, sem1)
        cd = pltpu.async_copy(dst_hbm.at[pl.ds(chunk0 + g, 2)], didx, dsem)
        cd.wait()
        c0.wait()
        pltpu.sync_copy(buf0, acc.at[didx.at[0]], add=True)
        c1.wait()
        pltpu.sync_copy(buf1, acc.at[didx.at[1]], add=True)
        return 0

    lax.fori_loop(0, trips, body, 0)
    plsc.subcore_barrier()
    pltpu.sync_copy(acc.at[pl.ds(base, RPT)], out_hbm.at[c, pl.ds(base, RPT)])


def _tc0_body(x_ref, w_ref, degp_ref, dinv_ref, hws_ref, hwsb_ref):
    deg = degp_ref[0] + degp_ref[1] + 1.0
    dinv = lax.rsqrt(deg)
    dinv_ref[...] = dinv
    hw = jnp.dot(x_ref[...], w_ref[...], preferred_element_type=jnp.float32)
    hws = hw * dinv[0:N]
    hws_ref[...] = hws
    hwsb_ref[...] = hws.astype(jnp.bfloat16)


_tc0 = pl.pallas_call(
    _tc0_body,
    out_shape=(
        jax.ShapeDtypeStruct((NPAD, 1), jnp.float32),
        jax.ShapeDtypeStruct((N, DH), jnp.float32),
        jax.ShapeDtypeStruct((N, DH), jnp.bfloat16),
    ),
)


def _tc_mid_body(acc_ref, hws_ref, dinv_ref, b_ref, w_ref, out_ref, outb_ref):
    dinv = dinv_ref[0:N]
    agg = (acc_ref[0, 0:N].astype(jnp.float32)
           + acc_ref[1, 0:N].astype(jnp.float32) + hws_ref[...])
    h = jnp.maximum(dinv * agg + b_ref[...], 0.0)
    hw = jnp.dot(h, w_ref[...], preferred_element_type=jnp.float32)
    hws = hw * dinv
    out_ref[...] = hws
    outb_ref[...] = hws.astype(jnp.bfloat16)


_tc_mid = pl.pallas_call(
    _tc_mid_body,
    out_shape=(
        jax.ShapeDtypeStruct((N, DH), jnp.float32),
        jax.ShapeDtypeStruct((N, DH), jnp.bfloat16),
    ),
)


def _tc_fin_body(acc_ref, hws_ref, dinv_ref, b_ref, batch_ref,
                 l1w_ref, l1b_ref, l2w_ref, l2b_ref, logp_ref, logits_ref):
    dinv = dinv_ref[0:N]
    agg = (acc_ref[0, 0:N].astype(jnp.float32)
           + acc_ref[1, 0:N].astype(jnp.float32) + hws_ref[...])
    h = jnp.maximum(dinv * agg + b_ref[...], 0.0)
    classes = lax.broadcasted_iota(jnp.int32, (B, N), 0)
    oht = (batch_ref[...] == classes).astype(jnp.float32)
    sums = jnp.dot(oht, h, preferred_element_type=jnp.float32)
    cnts = jnp.dot(oht, jnp.ones((N, 1), jnp.float32),
                   preferred_element_type=jnp.float32)
    pooled = sums / jnp.maximum(cnts, 1.0)
    h1 = jnp.dot(pooled, l1w_ref[...], preferred_element_type=jnp.float32)
    h1 = jnp.maximum(h1 + l1b_ref[...], 0.0)
    logits = jnp.dot(h1, l2w_ref[...],
                     preferred_element_type=jnp.float32) + l2b_ref[...]
    m = jnp.max(logits, axis=1, keepdims=True)
    lse = jnp.log(jnp.sum(jnp.exp(logits - m), axis=1, keepdims=True)) + m
    logp_ref[...] = logits - lse
    logits_ref[...] = logits


_tc_fin = pl.pallas_call(
    _tc_fin_body,
    out_shape=(
        jax.ShapeDtypeStruct((B, C), jnp.float32),
        jax.ShapeDtypeStruct((B, C), jnp.float32),
    ),
)


def kernel(x, edge_index, batch, W0, b0, W1, b1, W2, b2,
           lin1_w, lin1_b, lin2_w, lin2_b):
    pad = TOTCH_PAD * K - E
    src3 = jnp.concatenate(
        [edge_index[0], jnp.zeros((pad,), jnp.int32)]).reshape(TOTCH_PAD, K)
    dst3 = jnp.concatenate(
        [edge_index[1], jnp.full((pad,), N, jnp.int32)]).reshape(TOTCH_PAD, K)

    degp = _deg_kernel(dst3).reshape(NC, NPAD, 1)
    dinv, hws, hwsb = _tc0(x, W0, degp)
    acc = _scatter_kernel(hwsb, src3, dst3)
    hws, hwsb = _tc_mid(acc, hws, dinv, b0.reshape(1, DH), W1)
    acc = _scatter_kernel(hwsb, src3, dst3)
    hws, hwsb = _tc_mid(acc, hws, dinv, b1.reshape(1, DH), W2)
    acc = _scatter_kernel(hwsb, src3, dst3)
    logp, logits = _tc_fin(acc, hws, dinv, b2.reshape(1, DH),
                           batch.reshape(1, N).astype(jnp.int32),
                           lin1_w, lin1_b.reshape(1, DH),
                           lin2_w, lin2_b.reshape(1, C))
    return (logp, logits, logits)


# speedup vs baseline: 1.7917x; 1.0102x over previous
import functools

import jax
import jax.numpy as jnp
from jax import lax
from jax.experimental import pallas as pl
from jax.experimental.pallas import tpu as pltpu
from jax.experimental.pallas import tpu_sc as plsc

N = 10000
E = 320000
DH = 128
B = 64
C = 10

NC = 2
NS = 16
NW = NC * NS
K = 128
CHA = 80
CHB = 80
CH_MAX = max(CHA, CHB)
TOTCH = NS * (CHA + CHB)
TOTCH_PAD = TOTCH + CH_MAX
DEG_CH = TOTCH // NW
E_PAD = TOTCH * K
NPAD = 10240
RPT = NPAD // NS

_mesh = plsc.VectorSubcoreMesh(core_axis_name="c", subcore_axis_name="s")
_sc_params = pltpu.CompilerParams(needs_layout_passes=False)


@functools.partial(
    pl.kernel,
    out_type=jax.ShapeDtypeStruct((NC, NPAD), jnp.float32),
    scratch_types=[
        pltpu.VMEM((DEG_CH, K), jnp.int32),
        pltpu.VMEM((NPAD,), jnp.float32),
        pltpu.VMEM((NS, RPT), jnp.float32),
        pltpu.VMEM_SHARED((NS, NPAD), jnp.float32),
    ],
    mesh=_mesh,
    compiler_params=_sc_params,
)
def _deg_kernel(dst_hbm, out_hbm, dst_v, hist_v, red_v, shared):
    c = lax.axis_index("c")
    s = lax.axis_index("s")
    wid = c * NS + s
    pltpu.sync_copy(dst_hbm.at[pl.ds(wid * DEG_CH, DEG_CH)], dst_v)

    def zero_body(i, _):
        hist_v[pl.ds(i * 16, 16)] = jnp.zeros((16,), jnp.float32)
        return 0

    lax.fori_loop(0, NPAD // 16, zero_body, 0)

    ones = jnp.ones((16,), jnp.float32)

    def hist_body(j, _):
        for k in range(K // 16):
            idx = dst_v[j, pl.ds(k * 16, 16)]
            plsc.addupdate_scatter(hist_v, [idx], ones)
        return 0

    lax.fori_loop(0, DEG_CH, hist_body, 0)

    pltpu.sync_copy(hist_v, shared.at[s])
    plsc.subcore_barrier()
    pltpu.sync_copy(shared.at[:, pl.ds(s * RPT, RPT)], red_v)

    def red_body(k, _):
        v = red_v[0, pl.ds(k * 16, 16)]
        for r in range(1, NS):
            v = v + red_v[r, pl.ds(k * 16, 16)]
        hist_v[pl.ds(k * 16, 16)] = v
        return 0

    lax.fori_loop(0, RPT // 16, red_body, 0)
    pltpu.sync_copy(hist_v.at[pl.ds(0, RPT)], out_hbm.at[c, pl.ds(s * RPT, RPT)])


@functools.partial(
    pl.kernel,
    out_type=jax.ShapeDtypeStruct((NC, NPAD, DH), jnp.bfloat16),
    scratch_types=[
        pltpu.VMEM((CH_MAX, K), jnp.int32),
        pltpu.VMEM((4, K), jnp.int32),
        pltpu.VMEM((K, DH), jnp.bfloat16),
        pltpu.VMEM((K, DH), jnp.bfloat16),
        pltpu.VMEM((K, DH), jnp.bfloat16),
        pltpu.VMEM((K, DH), jnp.bfloat16),
        pltpu.VMEM_SHARED((NPAD, DH), jnp.bfloat16),
        pltpu.SemaphoreType.DMA,
        pltpu.SemaphoreType.DMA,
        pltpu.SemaphoreType.DMA,
        pltpu.SemaphoreType.DMA,
        pltpu.SemaphoreType.DMA,
    ],
    mesh=_mesh,
    compiler_params=pltpu.CompilerParams(
        needs_layout_passes=False, use_tc_tiling_on_sc=False),
)
def _scatter_kernel(hws_hbm, src_hbm, dst_hbm, out_hbm,
                    src_v, didx, buf0, buf1, buf2, buf3, acc,
                    sem0, sem1, sem2, sem3, dsem):
    c = lax.axis_index("c")
    s = lax.axis_index("s")
    chunk0 = jnp.where(c == 0, s * CHA, NS * CHA + s * CHB)
    trips = jnp.where(c == 0, CHA // 4, CHB // 4)
    pltpu.sync_copy(src_hbm.at[pl.ds(chunk0, CH_MAX)], src_v)

    def zero_body(i, _):
        for k in range(DH // 32):
            buf0[i, pl.ds(k * 32, 32)] = jnp.zeros((32,), jnp.bfloat16)
        return 0

    lax.fori_loop(0, K, zero_body, 0)
    base = s * RPT
    for i in range(RPT // K):
        pltpu.sync_copy(buf0, acc.at[pl.ds(base + i * K, K)])
    plsc.subcore_barrier()

    def body(i, _):
        g = 4 * i
        c0 = pltpu.async_copy(hws_hbm.at[src_v.at[g]], buf0, sem0)
        c1 = pltpu.async_copy(hws_hbm.at[src_v.at[g + 1]], buf1, sem1)
        c2 = pltpu.async_copy(hws_hbm.at[src_v.at[g + 2]], buf2, sem2)
        c3 = pltpu.async_copy(hws_hbm.at[src_v.at[g + 3]], buf3, sem3)
        cd = pltpu.async_copy(dst_hbm.at[pl.ds(chunk0 + g, 4)], didx, dsem)
        cd.wait()
        c0.wait()
        pltpu.sync_copy(buf0, acc.at[didx.at[0]], add=True)
        c1.wait()
        pltpu.sync_copy(buf1, acc.at[didx.at[1]], add=True)
        c2.wait()
        pltpu.sync_copy(buf2, acc.at[didx.at[2]], add=True)
        c3.wait()
        pltpu.sync_copy(buf3, acc.at[didx.at[3]], add=True)
        return 0

    lax.fori_loop(0, trips, body, 0)
    plsc.subcore_barrier()
    pltpu.sync_copy(acc.at[pl.ds(base, RPT)], out_hbm.at[c, pl.ds(base, RPT)])


def _tc0_body(x_ref, w_ref, degp_ref, dinv_ref, hws_ref, hwsb_ref):
    deg = degp_ref[0] + degp_ref[1] + 1.0
    dinv = lax.rsqrt(deg)
    dinv_ref[...] = dinv
    hw = jnp.dot(x_ref[...], w_ref[...], preferred_element_type=jnp.float32)
    hws = hw * dinv[0:N]
    hws_ref[...] = hws
    hwsb_ref[...] = hws.astype(jnp.bfloat16)


_tc0 = pl.pallas_call(
    _tc0_body,
    out_shape=(
        jax.ShapeDtypeStruct((NPAD, 1), jnp.float32),
        jax.ShapeDtypeStruct((N, DH), jnp.float32),
        jax.ShapeDtypeStruct((N, DH), jnp.bfloat16),
    ),
)


def _tc_mid_body(acc_ref, hws_ref, dinv_ref, b_ref, w_ref, out_ref, outb_ref):
    dinv = dinv_ref[0:N]
    agg = (acc_ref[0, 0:N].astype(jnp.float32)
           + acc_ref[1, 0:N].astype(jnp.float32) + hws_ref[...])
    h = jnp.maximum(dinv * agg + b_ref[...], 0.0)
    hw = jnp.dot(h, w_ref[...], preferred_element_type=jnp.float32)
    hws = hw * dinv
    out_ref[...] = hws
    outb_ref[...] = hws.astype(jnp.bfloat16)


_tc_mid = pl.pallas_call(
    _tc_mid_body,
    out_shape=(
        jax.ShapeDtypeStruct((N, DH), jnp.float32),
        jax.ShapeDtypeStruct((N, DH), jnp.bfloat16),
    ),
)


def _tc_fin_body(acc_ref, hws_ref, dinv_ref, b_ref, batch_ref,
                 l1w_ref, l1b_ref, l2w_ref, l2b_ref, logp_ref, logits_ref):
    dinv = dinv_ref[0:N]
    agg = (acc_ref[0, 0:N].astype(jnp.float32)
           + acc_ref[1, 0:N].astype(jnp.float32) + hws_ref[...])
    h = jnp.maximum(dinv * agg + b_ref[...], 0.0)
    classes = lax.broadcasted_iota(jnp.int32, (B, N), 0)
    oht = (batch_ref[...] == classes).astype(jnp.float32)
    sums = jnp.dot(oht, h, preferred_element_type=jnp.float32)
    cnts = jnp.dot(oht, jnp.ones((N, 1), jnp.float32),
                   preferred_element_type=jnp.float32)
    pooled = sums / jnp.maximum(cnts, 1.0)
    h1 = jnp.dot(pooled, l1w_ref[...], preferred_element_type=jnp.float32)
    h1 = jnp.maximum(h1 + l1b_ref[...], 0.0)
    logits = jnp.dot(h1, l2w_ref[...],
                     preferred_element_type=jnp.float32) + l2b_ref[...]
    m = jnp.max(logits, axis=1, keepdims=True)
    lse = jnp.log(jnp.sum(jnp.exp(logits - m), axis=1, keepdims=True)) + m
    logp_ref[...] = logits - lse
    logits_ref[...] = logits


_tc_fin = pl.pallas_call(
    _tc_fin_body,
    out_shape=(
        jax.ShapeDtypeStruct((B, C), jnp.float32),
        jax.ShapeDtypeStruct((B, C), jnp.float32),
    ),
)


def kernel(x, edge_index, batch, W0, b0, W1, b1, W2, b2,
           lin1_w, lin1_b, lin2_w, lin2_b):
    pad = TOTCH_PAD * K - E
    src3 = jnp.concatenate(
        [edge_index[0], jnp.zeros((pad,), jnp.int32)]).reshape(TOTCH_PAD, K)
    dst3 = jnp.concatenate(
        [edge_index[1], jnp.full((pad,), N, jnp.int32)]).reshape(TOTCH_PAD, K)

    degp = _deg_kernel(dst3).reshape(NC, NPAD, 1)
    dinv, hws, hwsb = _tc0(x, W0, degp)
    acc = _scatter_kernel(hwsb, src3, dst3)
    hws, hwsb = _tc_mid(acc, hws, dinv, b0.reshape(1, DH), W1)
    acc = _scatter_kernel(hwsb, src3, dst3)
    hws, hwsb = _tc_mid(acc, hws, dinv, b1.reshape(1, DH), W2)
    acc = _scatter_kernel(hwsb, src3, dst3)
    logp, logits = _tc_fin(acc, hws, dinv, b2.reshape(1, DH),
                           batch.reshape(1, N).astype(jnp.int32),
                           lin1_w, lin1_b.reshape(1, DH),
                           lin2_w, lin2_b.reshape(1, C))
    return (logp, logits, logits)


# speedup vs baseline: 3.8400x; 2.1433x over previous
import functools

import jax
import jax.numpy as jnp
from jax import lax
from jax.experimental import pallas as pl
from jax.experimental.pallas import tpu as pltpu
from jax.experimental.pallas import tpu_sc as plsc

N = 10000
E = 320000
DH = 128
B = 64
C = 10

NC = 2
NS = 16
NW = NC * NS
K = 128
CHA = 80
CHB = 80
CH_MAX = max(CHA, CHB)
TOTCH = NS * (CHA + CHB)
TOTCH_PAD = TOTCH + CH_MAX
DEG_CH = TOTCH // NW
E_PAD = TOTCH * K
NPAD = 10240
RPT = NPAD // NS

_mesh = plsc.VectorSubcoreMesh(core_axis_name="c", subcore_axis_name="s")
_sc_params = pltpu.CompilerParams(needs_layout_passes=False)


@functools.partial(
    pl.kernel,
    out_type=jax.ShapeDtypeStruct((NC, NPAD), jnp.float32),
    scratch_types=[
        pltpu.VMEM((DEG_CH, K), jnp.int32),
        pltpu.VMEM((NPAD,), jnp.float32),
        pltpu.VMEM((NS, RPT), jnp.float32),
        pltpu.VMEM_SHARED((NS, NPAD), jnp.float32),
    ],
    mesh=_mesh,
    compiler_params=_sc_params,
)
def _deg_kernel(dst_hbm, out_hbm, dst_v, hist_v, red_v, shared):
    c = lax.axis_index("c")
    s = lax.axis_index("s")
    wid = c * NS + s
    pltpu.sync_copy(dst_hbm.at[pl.ds(wid * DEG_CH, DEG_CH)], dst_v)

    def zero_body(i, _):
        hist_v[pl.ds(i * 16, 16)] = jnp.zeros((16,), jnp.float32)
        return 0

    lax.fori_loop(0, NPAD // 16, zero_body, 0)

    ones = jnp.ones((16,), jnp.float32)

    def hist_body(j, _):
        for k in range(K // 16):
            idx = dst_v[j, pl.ds(k * 16, 16)]
            plsc.addupdate_scatter(hist_v, [idx], ones)
        return 0

    lax.fori_loop(0, DEG_CH, hist_body, 0)

    pltpu.sync_copy(hist_v, shared.at[s])
    plsc.subcore_barrier()
    pltpu.sync_copy(shared.at[:, pl.ds(s * RPT, RPT)], red_v)

    def red_body(k, _):
        v = red_v[0, pl.ds(k * 16, 16)]
        for r in range(1, NS):
            v = v + red_v[r, pl.ds(k * 16, 16)]
        hist_v[pl.ds(k * 16, 16)] = v
        return 0

    lax.fori_loop(0, RPT // 16, red_body, 0)
    pltpu.sync_copy(hist_v.at[pl.ds(0, RPT)], out_hbm.at[c, pl.ds(s * RPT, RPT)])


@functools.partial(
    pl.kernel,
    out_type=jax.ShapeDtypeStruct((NC, NPAD, DH), jnp.bfloat16),
    scratch_types=[
        pltpu.VMEM((CH_MAX, K), jnp.int32),
        pltpu.VMEM((4, K), jnp.int32),
        pltpu.VMEM((K, DH), jnp.bfloat16),
        pltpu.VMEM((K, DH), jnp.bfloat16),
        pltpu.VMEM((K, DH), jnp.bfloat16),
        pltpu.VMEM((K, DH), jnp.bfloat16),
        pltpu.VMEM_SHARED((NPAD, DH), jnp.bfloat16),
        pltpu.VMEM_SHARED((NPAD, DH), jnp.bfloat16),
        pltpu.SemaphoreType.DMA,
        pltpu.SemaphoreType.DMA,
        pltpu.SemaphoreType.DMA,
        pltpu.SemaphoreType.DMA,
        pltpu.SemaphoreType.DMA,
    ],
    mesh=_mesh,
    compiler_params=pltpu.CompilerParams(
        needs_layout_passes=False, use_tc_tiling_on_sc=False),
)
def _scatter_kernel(hws_hbm, src_hbm, dst_hbm, out_hbm,
                    src_v, didx, buf0, buf1, buf2, buf3, acc, hws_s,
                    sem0, sem1, sem2, sem3, dsem):
    c = lax.axis_index("c")
    s = lax.axis_index("s")
    chunk0 = jnp.where(c == 0, s * CHA, NS * CHA + s * CHB)
    trips = jnp.where(c == 0, CHA // 4, CHB // 4)
    pltpu.sync_copy(src_hbm.at[pl.ds(chunk0, CH_MAX)], src_v)
    pltpu.sync_copy(hws_hbm.at[pl.ds(s * RPT, RPT)],
                    hws_s.at[pl.ds(s * RPT, RPT)])

    def zero_body(i, _):
        for k in range(DH // 32):
            buf0[i, pl.ds(k * 32, 32)] = jnp.zeros((32,), jnp.bfloat16)
        return 0

    lax.fori_loop(0, K, zero_body, 0)
    base = s * RPT
    for i in range(RPT // K):
        pltpu.sync_copy(buf0, acc.at[pl.ds(base + i * K, K)])
    plsc.subcore_barrier()

    def body(i, _):
        g = 4 * i
        c0 = pltpu.async_copy(hws_s.at[src_v.at[g]], buf0, sem0)
        c1 = pltpu.async_copy(hws_s.at[src_v.at[g + 1]], buf1, sem1)
        c2 = pltpu.async_copy(hws_s.at[src_v.at[g + 2]], buf2, sem2)
        c3 = pltpu.async_copy(hws_s.at[src_v.at[g + 3]], buf3, sem3)
        cd = pltpu.async_copy(dst_hbm.at[pl.ds(chunk0 + g, 4)], didx, dsem)
        cd.wait()
        c0.wait()
        pltpu.sync_copy(buf0, acc.at[didx.at[0]], add=True)
        c1.wait()
        pltpu.sync_copy(buf1, acc.at[didx.at[1]], add=True)
        c2.wait()
        pltpu.sync_copy(buf2, acc.at[didx.at[2]], add=True)
        c3.wait()
        pltpu.sync_copy(buf3, acc.at[didx.at[3]], add=True)
        return 0

    lax.fori_loop(0, trips, body, 0)
    plsc.subcore_barrier()
    pltpu.sync_copy(acc.at[pl.ds(base, RPT)], out_hbm.at[c, pl.ds(base, RPT)])


def _tc0_body(x_ref, w_ref, degp_ref, dinv_ref, hws_ref, hwsb_ref):
    deg = degp_ref[0] + degp_ref[1] + 1.0
    dinv = lax.rsqrt(deg)
    dinv_ref[...] = dinv
    hw = jnp.dot(x_ref[...], w_ref[...], preferred_element_type=jnp.float32)
    hws = hw * dinv[0:N]
    hws_ref[...] = hws
    hwsb_ref[0:N] = hws.astype(jnp.bfloat16)
    hwsb_ref[N:NPAD] = jnp.zeros((NPAD - N, DH), jnp.bfloat16)


_tc0 = pl.pallas_call(
    _tc0_body,
    out_shape=(
        jax.ShapeDtypeStruct((NPAD, 1), jnp.float32),
        jax.ShapeDtypeStruct((N, DH), jnp.float32),
        jax.ShapeDtypeStruct((NPAD, DH), jnp.bfloat16),
    ),
)


def _tc_mid_body(acc_ref, hws_ref, dinv_ref, b_ref, w_ref, out_ref, outb_ref):
    dinv = dinv_ref[0:N]
    agg = (acc_ref[0, 0:N].astype(jnp.float32)
           + acc_ref[1, 0:N].astype(jnp.float32) + hws_ref[...])
    h = jnp.maximum(dinv * agg + b_ref[...], 0.0)
    hw = jnp.dot(h, w_ref[...], preferred_element_type=jnp.float32)
    hws = hw * dinv
    out_ref[...] = hws
    outb_ref[0:N] = hws.astype(jnp.bfloat16)
    outb_ref[N:NPAD] = jnp.zeros((NPAD - N, DH), jnp.bfloat16)


_tc_mid = pl.pallas_call(
    _tc_mid_body,
    out_shape=(
        jax.ShapeDtypeStruct((N, DH), jnp.float32),
        jax.ShapeDtypeStruct((NPAD, DH), jnp.bfloat16),
    ),
)


def _tc_fin_body(acc_ref, hws_ref, dinv_ref, b_ref, batch_ref,
                 l1w_ref, l1b_ref, l2w_ref, l2b_ref, logp_ref, logits_ref):
    dinv = dinv_ref[0:N]
    agg = (acc_ref[0, 0:N].astype(jnp.float32)
           + acc_ref[1, 0:N].astype(jnp.float32) + hws_ref[...])
    h = jnp.maximum(dinv * agg + b_ref[...], 0.0)
    classes = lax.broadcasted_iota(jnp.int32, (B, N), 0)
    oht = (batch_ref[...] == classes).astype(jnp.float32)
    sums = jnp.dot(oht, h, preferred_element_type=jnp.float32)
    cnts = jnp.dot(oht, jnp.ones((N, 1), jnp.float32),
                   preferred_element_type=jnp.float32)
    pooled = sums / jnp.maximum(cnts, 1.0)
    h1 = jnp.dot(pooled, l1w_ref[...], preferred_element_type=jnp.float32)
    h1 = jnp.maximum(h1 + l1b_ref[...], 0.0)
    logits = jnp.dot(h1, l2w_ref[...],
                     preferred_element_type=jnp.float32) + l2b_ref[...]
    m = jnp.max(logits, axis=1, keepdims=True)
    lse = jnp.log(jnp.sum(jnp.exp(logits - m), axis=1, keepdims=True)) + m
    logp_ref[...] = logits - lse
    logits_ref[...] = logits


_tc_fin = pl.pallas_call(
    _tc_fin_body,
    out_shape=(
        jax.ShapeDtypeStruct((B, C), jnp.float32),
        jax.ShapeDtypeStruct((B, C), jnp.float32),
    ),
)


def kernel(x, edge_index, batch, W0, b0, W1, b1, W2, b2,
           lin1_w, lin1_b, lin2_w, lin2_b):
    pad = TOTCH_PAD * K - E
    src3 = jnp.concatenate(
        [edge_index[0], jnp.zeros((pad,), jnp.int32)]).reshape(TOTCH_PAD, K)
    dst3 = jnp.concatenate(
        [edge_index[1], jnp.full((pad,), N, jnp.int32)]).reshape(TOTCH_PAD, K)

    degp = _deg_kernel(dst3).reshape(NC, NPAD, 1)
    dinv, hws, hwsb = _tc0(x, W0, degp)
    acc = _scatter_kernel(hwsb, src3, dst3)
    hws, hwsb = _tc_mid(acc, hws, dinv, b0.reshape(1, DH), W1)
    acc = _scatter_kernel(hwsb, src3, dst3)
    hws, hwsb = _tc_mid(acc, hws, dinv, b1.reshape(1, DH), W2)
    acc = _scatter_kernel(hwsb, src3, dst3)
    logp, logits = _tc_fin(acc, hws, dinv, b2.reshape(1, DH),
                           batch.reshape(1, N).astype(jnp.int32),
                           lin1_w, lin1_b.reshape(1, DH),
                           lin2_w, lin2_b.reshape(1, C))
    return (logp, logits, logits)


# speedup vs baseline: 4.1364x; 1.0772x over previous
import functools

import jax
import jax.numpy as jnp
from jax import lax
from jax.experimental import pallas as pl
from jax.experimental.pallas import tpu as pltpu
from jax.experimental.pallas import tpu_sc as plsc

N = 10000
E = 320000
DH = 128
B = 64
C = 10

NC = 2
NS = 16
NW = NC * NS
K = 128
CHA = 92
CHB = 68
CH_MAX = max(CHA, CHB)
TOTCH = NS * (CHA + CHB)
TOTCH_PAD = TOTCH + CH_MAX
DEG_CH = TOTCH // NW
E_PAD = TOTCH * K
NPAD = 10240
RPT = NPAD // NS

_mesh = plsc.VectorSubcoreMesh(core_axis_name="c", subcore_axis_name="s")
_sc_params = pltpu.CompilerParams(needs_layout_passes=False)


@functools.partial(
    pl.kernel,
    out_type=jax.ShapeDtypeStruct((NC, NPAD), jnp.float32),
    scratch_types=[
        pltpu.VMEM((DEG_CH, K), jnp.int32),
        pltpu.VMEM((NPAD,), jnp.float32),
        pltpu.VMEM((NS, RPT), jnp.float32),
        pltpu.VMEM_SHARED((NS, NPAD), jnp.float32),
    ],
    mesh=_mesh,
    compiler_params=_sc_params,
)
def _deg_kernel(dst_hbm, out_hbm, dst_v, hist_v, red_v, shared):
    c = lax.axis_index("c")
    s = lax.axis_index("s")
    wid = c * NS + s
    pltpu.sync_copy(dst_hbm.at[pl.ds(wid * DEG_CH, DEG_CH)], dst_v)

    def zero_body(i, _):
        hist_v[pl.ds(i * 16, 16)] = jnp.zeros((16,), jnp.float32)
        return 0

    lax.fori_loop(0, NPAD // 16, zero_body, 0)

    ones = jnp.ones((16,), jnp.float32)

    def hist_body(j, _):
        for k in range(K // 16):
            idx = dst_v[j, pl.ds(k * 16, 16)]
            plsc.addupdate_scatter(hist_v, [idx], ones)
        return 0

    lax.fori_loop(0, DEG_CH, hist_body, 0)

    pltpu.sync_copy(hist_v, shared.at[s])
    plsc.subcore_barrier()
    pltpu.sync_copy(shared.at[:, pl.ds(s * RPT, RPT)], red_v)

    def red_body(k, _):
        v = red_v[0, pl.ds(k * 16, 16)]
        for r in range(1, NS):
            v = v + red_v[r, pl.ds(k * 16, 16)]
        hist_v[pl.ds(k * 16, 16)] = v
        return 0

    lax.fori_loop(0, RPT // 16, red_body, 0)
    pltpu.sync_copy(hist_v.at[pl.ds(0, RPT)], out_hbm.at[c, pl.ds(s * RPT, RPT)])


@functools.partial(
    pl.kernel,
    out_type=jax.ShapeDtypeStruct((NC, NPAD, DH), jnp.bfloat16),
    scratch_types=[
        pltpu.VMEM((CH_MAX, K), jnp.int32),
        pltpu.VMEM((4, K), jnp.int32),
        pltpu.VMEM((K, DH), jnp.bfloat16),
        pltpu.VMEM((K, DH), jnp.bfloat16),
        pltpu.VMEM((K, DH), jnp.bfloat16),
        pltpu.VMEM((K, DH), jnp.bfloat16),
        pltpu.VMEM_SHARED((NPAD, DH), jnp.bfloat16),
        pltpu.VMEM_SHARED((NPAD, DH), jnp.bfloat16),
        pltpu.SemaphoreType.DMA,
        pltpu.SemaphoreType.DMA,
        pltpu.SemaphoreType.DMA,
        pltpu.SemaphoreType.DMA,
        pltpu.SemaphoreType.DMA,
    ],
    mesh=_mesh,
    compiler_params=pltpu.CompilerParams(
        needs_layout_passes=False, use_tc_tiling_on_sc=False),
)
def _scatter_kernel(hws_hbm, src_hbm, dst_hbm, out_hbm,
                    src_v, didx, buf0, buf1, buf2, buf3, acc, hws_s,
                    sem0, sem1, sem2, sem3, dsem):
    c = lax.axis_index("c")
    s = lax.axis_index("s")
    chunk0 = jnp.where(c == 0, s * CHA, NS * CHA + s * CHB)
    trips = jnp.where(c == 0, CHA // 4, CHB // 4)
    cs = pltpu.async_copy(src_hbm.at[pl.ds(chunk0, CH_MAX)], src_v, sem1)
    ct = pltpu.async_copy(hws_hbm.at[pl.ds(s * RPT, RPT)],
                          hws_s.at[pl.ds(s * RPT, RPT)], sem0)

    def zero_body(i, _):
        for k in range(DH // 32):
            buf0[i, pl.ds(k * 32, 32)] = jnp.zeros((32,), jnp.bfloat16)
        return 0

    lax.fori_loop(0, K, zero_body, 0)
    base = s * RPT
    for i in range(RPT // K):
        pltpu.sync_copy(buf0, acc.at[pl.ds(base + i * K, K)])
    ct.wait()
    cs.wait()
    plsc.subcore_barrier()

    def body(i, _):
        g = 4 * i
        c0 = pltpu.async_copy(hws_s.at[src_v.at[g]], buf0, sem0)
        c1 = pltpu.async_copy(hws_s.at[src_v.at[g + 1]], buf1, sem1)
        c2 = pltpu.async_copy(hws_s.at[src_v.at[g + 2]], buf2, sem2)
        c3 = pltpu.async_copy(hws_s.at[src_v.at[g + 3]], buf3, sem3)
        cd = pltpu.async_copy(dst_hbm.at[pl.ds(chunk0 + g, 4)], didx, dsem)
        cd.wait()
        c0.wait()
        pltpu.sync_copy(buf0, acc.at[didx.at[0]], add=True)
        c1.wait()
        pltpu.sync_copy(buf1, acc.at[didx.at[1]], add=True)
        c2.wait()
        pltpu.sync_copy(buf2, acc.at[didx.at[2]], add=True)
        c3.wait()
        pltpu.sync_copy(buf3, acc.at[didx.at[3]], add=True)
        return 0

    lax.fori_loop(0, trips, body, 0)
    plsc.subcore_barrier()
    pltpu.sync_copy(acc.at[pl.ds(base, RPT)], out_hbm.at[c, pl.ds(base, RPT)])


def _tc0_body(x_ref, w_ref, degp_ref, dinv_ref, hws_ref, hwsb_ref):
    deg = degp_ref[0] + degp_ref[1] + 1.0
    dinv = lax.rsqrt(deg)
    dinv_ref[...] = dinv
    hw = jnp.dot(x_ref[...], w_ref[...], preferred_element_type=jnp.float32)
    hws = hw * dinv[0:N]
    hws_ref[...] = hws
    hwsb_ref[0:N] = hws.astype(jnp.bfloat16)
    hwsb_ref[N:NPAD] = jnp.zeros((NPAD - N, DH), jnp.bfloat16)


_tc0 = pl.pallas_call(
    _tc0_body,
    out_shape=(
        jax.ShapeDtypeStruct((NPAD, 1), jnp.float32),
        jax.ShapeDtypeStruct((N, DH), jnp.float32),
        jax.ShapeDtypeStruct((NPAD, DH), jnp.bfloat16),
    ),
)


def _tc_mid_body(acc_ref, hws_ref, dinv_ref, b_ref, w_ref, out_ref, outb_ref):
    dinv = dinv_ref[0:N]
    agg = (acc_ref[0, 0:N].astype(jnp.float32)
           + acc_ref[1, 0:N].astype(jnp.float32) + hws_ref[...])
    h = jnp.maximum(dinv * agg + b_ref[...], 0.0)
    hw = jnp.dot(h, w_ref[...], preferred_element_type=jnp.float32)
    hws = hw * dinv
    out_ref[...] = hws
    outb_ref[0:N] = hws.astype(jnp.bfloat16)
    outb_ref[N:NPAD] = jnp.zeros((NPAD - N, DH), jnp.bfloat16)


_tc_mid = pl.pallas_call(
    _tc_mid_body,
    out_shape=(
        jax.ShapeDtypeStruct((N, DH), jnp.float32),
        jax.ShapeDtypeStruct((NPAD, DH), jnp.bfloat16),
    ),
)


def _tc_fin_body(acc_ref, hws_ref, dinv_ref, b_ref, batch_ref,
                 l1w_ref, l1b_ref, l2w_ref, l2b_ref, logp_ref, logits_ref):
    dinv = dinv_ref[0:N]
    agg = (acc_ref[0, 0:N].astype(jnp.float32)
           + acc_ref[1, 0:N].astype(jnp.float32) + hws_ref[...])
    h = jnp.maximum(dinv * agg + b_ref[...], 0.0)
    classes = lax.broadcasted_iota(jnp.int32, (B, N), 0)
    oht = (batch_ref[...] == classes).astype(jnp.float32)
    sums = jnp.dot(oht, h, preferred_element_type=jnp.float32)
    cnts = jnp.dot(oht, jnp.ones((N, 1), jnp.float32),
                   preferred_element_type=jnp.float32)
    pooled = sums / jnp.maximum(cnts, 1.0)
    h1 = jnp.dot(pooled, l1w_ref[...], preferred_element_type=jnp.float32)
    h1 = jnp.maximum(h1 + l1b_ref[...], 0.0)
    logits = jnp.dot(h1, l2w_ref[...],
                     preferred_element_type=jnp.float32) + l2b_ref[...]
    m = jnp.max(logits, axis=1, keepdims=True)
    lse = jnp.log(jnp.sum(jnp.exp(logits - m), axis=1, keepdims=True)) + m
    logp_ref[...] = logits - lse
    logits_ref[...] = logits


_tc_fin = pl.pallas_call(
    _tc_fin_body,
    out_shape=(
        jax.ShapeDtypeStruct((B, C), jnp.float32),
        jax.ShapeDtypeStruct((B, C), jnp.float32),
    ),
)


def kernel(x, edge_index, batch, W0, b0, W1, b1, W2, b2,
           lin1_w, lin1_b, lin2_w, lin2_b):
    pad = TOTCH_PAD * K - E
    src3 = jnp.concatenate(
        [edge_index[0], jnp.zeros((pad,), jnp.int32)]).reshape(TOTCH_PAD, K)
    dst3 = jnp.concatenate(
        [edge_index[1], jnp.full((pad,), N, jnp.int32)]).reshape(TOTCH_PAD, K)

    degp = _deg_kernel(dst3).reshape(NC, NPAD, 1)
    dinv, hws, hwsb = _tc0(x, W0, degp)
    acc = _scatter_kernel(hwsb, src3, dst3)
    hws, hwsb = _tc_mid(acc, hws, dinv, b0.reshape(1, DH), W1)
    acc = _scatter_kernel(hwsb, src3, dst3)
    hws, hwsb = _tc_mid(acc, hws, dinv, b1.reshape(1, DH), W2)
    acc = _scatter_kernel(hwsb, src3, dst3)
    logp, logits = _tc_fin(acc, hws, dinv, b2.reshape(1, DH),
                           batch.reshape(1, N).astype(jnp.int32),
                           lin1_w, lin1_b.reshape(1, DH),
                           lin2_w, lin2_b.reshape(1, C))
    return (logp, logits, logits)
